# baseline (speedup 1.0000x reference)
import functools

import jax
import jax.numpy as jnp
from jax import lax
from jax.experimental import pallas as pl
from jax.experimental.pallas import tpu as pltpu
from jax.experimental.pallas import tpu_sc as plsc

N_NODES = 10000
D_FEAT = 128
HIDDEN = 16
N_DRUGS = 1024

NC = 2
NS = 16
NW = NC * NS
L = 16

G = 8
N_PAD = 10240
NG = N_PAD // G
NG_REAL = N_NODES // G
RPT = N_PAD // NS
GPT = NG // NS
CHUNK = 2000
NBUF = 3
NSEM = 8


def _sc_mesh():
    return plsc.VectorSubcoreMesh(
        core_axis_name="c", subcore_axis_name="s", num_cores=NC, num_subcores=NS
    )


_SC_PARAMS = pltpu.CompilerParams(use_tc_tiling_on_sc=False)


def _make_deg_kernel(k_per_tile):
    @functools.partial(
        pl.kernel,
        out_type=jax.ShapeDtypeStruct((NC, N_PAD, L), jnp.float32),
        mesh=_sc_mesh(),
        scratch_types=[
            pltpu.VMEM((k_per_tile, CHUNK), jnp.int32),
            pltpu.VMEM((CHUNK, L), jnp.float32),
            pltpu.VMEM_SHARED((N_PAD, L), jnp.float32),
            pltpu.SemaphoreType.DMA((NSEM,)),
        ],
        compiler_params=_SC_PARAMS,
    )
    def deg_kernel(ei3, ones_hbm, zeros_hbm, out, dst_v, ones_v, acc_sh, sems):
        cid = lax.axis_index("c")
        sid = lax.axis_index("s")
        wid = sid * NC + cid
        pltpu.sync_copy(zeros_hbm, acc_sh.at[pl.ds(sid * RPT, RPT)])
        pltpu.sync_copy(ei3.at[1, pl.ds(wid * k_per_tile, k_per_tile)], dst_v)
        pltpu.sync_copy(ones_hbm, ones_v)
        plsc.subcore_barrier()
        sd = []
        for j in range(k_per_tile):
            sd.append(pltpu.async_copy(
                ones_v, acc_sh.at[dst_v.at[j]], sems.at[j % NSEM], add=True
            ))
        for d in sd:
            d.wait()
        plsc.subcore_barrier()
        pltpu.sync_copy(
            acc_sh.at[pl.ds(sid * RPT, RPT)],
            out.at[cid, pl.ds(sid * RPT, RPT)],
        )

    return deg_kernel


def _make_edge_kernel(k_per_tile):
    @functools.partial(
        pl.kernel,
        out_type=jax.ShapeDtypeStruct((NC, N_PAD, L), jnp.float32),
        mesh=_sc_mesh(),
        scratch_types=[
            pltpu.VMEM((k_per_tile, CHUNK), jnp.int32),
            pltpu.VMEM((k_per_tile, CHUNK), jnp.int32),
            pltpu.VMEM((NBUF, CHUNK, L), jnp.float32),
            pltpu.VMEM_SHARED((N_PAD, L), jnp.float32),
            pltpu.SemaphoreType.DMA((NBUF,)),
            pltpu.SemaphoreType.DMA((NBUF,)),
        ],
        compiler_params=_SC_PARAMS,
    )
    def edge_kernel(y_hbm, ei3, zeros_hbm, out, src_v, dst_v, rows_v,
                    acc_sh, gsem, ssem):
        cid = lax.axis_index("c")
        sid = lax.axis_index("s")
        wid = sid * NC + cid
        pltpu.sync_copy(zeros_hbm, acc_sh.at[pl.ds(sid * RPT, RPT)])
        pltpu.sync_copy(ei3.at[0, pl.ds(wid * k_per_tile, k_per_tile)], src_v)
        pltpu.sync_copy(ei3.at[1, pl.ds(wid * k_per_tile, k_per_tile)], dst_v)
        plsc.subcore_barrier()
        gd = {}
        sd = {}
        for j in range(min(NBUF, k_per_tile)):
            gd[j] = pltpu.async_copy(
                y_hbm.at[src_v.at[j]], rows_v.at[j % NBUF], gsem.at[j % NBUF]
            )
        lag = NBUF // 2
        for j in range(k_per_tile):
            b = j % NBUF
            gd[j].wait()
            sd[j] = pltpu.async_copy(
                rows_v.at[b], acc_sh.at[dst_v.at[j]], ssem.at[b], add=True
            )
            jj = j - lag
            if jj >= 0 and jj + NBUF < k_per_tile:
                nj = jj + NBUF
                sd[jj].wait()
                gd[nj] = pltpu.async_copy(
                    y_hbm.at[src_v.at[nj]], rows_v.at[nj % NBUF],
                    gsem.at[nj % NBUF]
                )
        for j in range(max(k_per_tile - NBUF, 0), k_per_tile):
            sd[j].wait()
        plsc.subcore_barrier()
        pltpu.sync_copy(
            acc_sh.at[pl.ds(sid * RPT, RPT)],
            out.at[cid, pl.ds(sid * RPT, RPT)],
        )

    return edge_kernel


def _block_diag(w, reps):
    r, c = w.shape
    tiled = jnp.tile(w, (reps, reps))
    i0 = lax.broadcasted_iota(jnp.int32, (r * reps, c * reps), 0) // r
    i1 = lax.broadcasted_iota(jnp.int32, (r * reps, c * reps), 1) // c
    return jnp.where(i0 == i1, tiled, 0.0)


def _tca_body(xg_ref, w1_ref, xw_ref):
    bd = _block_diag(w1_ref[...], G)
    xw = jnp.dot(xg_ref[...], bd, preferred_element_type=jnp.float32)
    xw_ref[...] = jnp.concatenate(
        [xw, jnp.zeros((NG - NG_REAL, G * HIDDEN), jnp.float32)], axis=0
    )


def _tcb_body(degp_ref, xw_ref, dis_ref, y1_ref):
    dis = lax.rsqrt(degp_ref[0] + degp_ref[1] + 1.0)
    dis_ref[...] = dis
    y1_ref[...] = dis * xw_ref[...]


def _tc2_body(sp_ref, y1_ref, dis_ref, w2_ref, b1_ref, y2_ref):
    dis = dis_ref[...]
    h1 = jnp.maximum(dis * (sp_ref[0] + sp_ref[1] + y1_ref[...]) + b1_ref[...],
                     0.0)
    bd = _block_diag(w2_ref[...], G)
    y2_ref[...] = dis * jnp.dot(h1, bd, preferred_element_type=jnp.float32)


def _tc3_body(sp_ref, y2_ref, dis_ref, b2_ref, pred_ref, out_ref):
    h2_g = dis_ref[...] * (sp_ref[0] + sp_ref[1] + y2_ref[...]) + b2_ref[...]
    rows = lax.broadcasted_iota(jnp.int32, (N_DRUGS, N_DRUGS // G), 0)
    cols = lax.broadcasted_iota(jnp.int32, (N_DRUGS, N_DRUGS // G), 1)
    h2 = jnp.zeros((N_DRUGS, HIDDEN), jnp.float32)
    for i in range(G):
        e_i = jnp.where((rows % G == i) & (rows // G == cols), 1.0, 0.0)
        h2 = h2 + jnp.dot(e_i, h2_g[:, i * HIDDEN:(i + 1) * HIDDEN],
                          preferred_element_type=jnp.float32)
    t = jnp.dot(h2, pred_ref[...], preferred_element_type=jnp.float32)
    out_ref[...] = lax.dot_general(
        t, h2, (((1,), (1,)), ((), ())), preferred_element_type=jnp.float32
    )


def kernel(x, edge_index, number_of_drugs, W1, b1, W2, b2, predictor):
    del number_of_drugs
    e = edge_index.shape[1]
    k_per_tile = e // (NW * CHUNK)
    ei3 = edge_index.astype(jnp.int32).reshape(2, NW * k_per_tile, CHUNK)

    zeros = jnp.zeros((RPT, L), jnp.float32)
    ones = jnp.ones((CHUNK, L), jnp.float32)
    b1t = jnp.tile(b1, G).reshape(1, G * HIDDEN)
    b2t = jnp.tile(b2, G).reshape(1, G * HIDDEN)
    nd_g = N_DRUGS // G

    deg_kernel = _make_deg_kernel(k_per_tile)
    edge_kernel = _make_edge_kernel(k_per_tile)

    xw_g = pl.pallas_call(
        _tca_body,
        out_shape=jax.ShapeDtypeStruct((NG, G * HIDDEN), jnp.float32),
    )(x.reshape(NG_REAL, G * D_FEAT), W1)

    degp = deg_kernel(ei3, ones, zeros).reshape(NC, NG, G * HIDDEN)

    dis_g, y1_g = pl.pallas_call(
        _tcb_body,
        out_shape=[
            jax.ShapeDtypeStruct((NG, G * HIDDEN), jnp.float32),
            jax.ShapeDtypeStruct((NG, G * HIDDEN), jnp.float32),
        ],
    )(degp, xw_g)

    sp1 = edge_kernel(y1_g.reshape(N_PAD, L), ei3, zeros).reshape(
        NC, NG, G * HIDDEN)

    y2_g = pl.pallas_call(
        _tc2_body,
        out_shape=jax.ShapeDtypeStruct((NG, G * HIDDEN), jnp.float32),
    )(sp1, y1_g, dis_g, W2, b1t)

    sp2 = edge_kernel(y2_g.reshape(N_PAD, L), ei3, zeros)

    sp2_g = sp2.reshape(NC, NG, G * HIDDEN)
    scores = pl.pallas_call(
        _tc3_body,
        out_shape=jax.ShapeDtypeStruct((N_DRUGS, N_DRUGS), jnp.float32),
    )(sp2_g[:, :nd_g], y2_g[:nd_g], dis_g[:nd_g], b2t, predictor)

    return scores


# speedup vs baseline: 1.0809x; 1.0809x over previous
import functools

import jax
import jax.numpy as jnp
from jax import lax
from jax.experimental import pallas as pl
from jax.experimental.pallas import tpu as pltpu
from jax.experimental.pallas import tpu_sc as plsc

N_NODES = 10000
D_FEAT = 128
HIDDEN = 16
N_DRUGS = 1024

NC = 2
NS = 16
NW = NC * NS
L = 16

G = 8
N_PAD = 10240
NG = N_PAD // G
NG_REAL = N_NODES // G
RPT = N_PAD // NS
GPT = NG // NS
CHUNK = 1000
NBUF = 6
NSEM = 8


def _sc_mesh():
    return plsc.VectorSubcoreMesh(
        core_axis_name="c", subcore_axis_name="s", num_cores=NC, num_subcores=NS
    )


_SC_PARAMS = pltpu.CompilerParams(use_tc_tiling_on_sc=False)


def _make_deg_kernel(k_per_tile):
    @functools.partial(
        pl.kernel,
        out_type=jax.ShapeDtypeStruct((NC, N_PAD, L), jnp.float32),
        mesh=_sc_mesh(),
        scratch_types=[
            pltpu.VMEM((k_per_tile, CHUNK), jnp.int32),
            pltpu.VMEM((CHUNK, L), jnp.float32),
            pltpu.VMEM_SHARED((N_PAD, L), jnp.float32),
            pltpu.SemaphoreType.DMA((NSEM,)),
        ],
        compiler_params=_SC_PARAMS,
    )
    def deg_kernel(ei3, ones_hbm, zeros_hbm, out, dst_v, ones_v, acc_sh, sems):
        cid = lax.axis_index("c")
        sid = lax.axis_index("s")
        wid = sid * NC + cid
        pltpu.sync_copy(zeros_hbm, acc_sh.at[pl.ds(sid * RPT, RPT)])
        pltpu.sync_copy(ei3.at[1, pl.ds(wid * k_per_tile, k_per_tile)], dst_v)
        pltpu.sync_copy(ones_hbm, ones_v)
        plsc.subcore_barrier()
        sd = []
        for j in range(k_per_tile):
            sd.append(pltpu.async_copy(
                ones_v, acc_sh.at[dst_v.at[j]], sems.at[j % NSEM], add=True
            ))
        for d in sd:
            d.wait()
        plsc.subcore_barrier()
        pltpu.sync_copy(
            acc_sh.at[pl.ds(sid * RPT, RPT)],
            out.at[cid, pl.ds(sid * RPT, RPT)],
        )

    return deg_kernel


def _make_edge_kernel(k_per_tile):
    @functools.partial(
        pl.kernel,
        out_type=jax.ShapeDtypeStruct((NC, N_PAD, L), jnp.float32),
        mesh=_sc_mesh(),
        scratch_types=[
            pltpu.VMEM((k_per_tile, CHUNK), jnp.int32),
            pltpu.VMEM((k_per_tile, CHUNK), jnp.int32),
            pltpu.VMEM((NBUF, CHUNK, L), jnp.float32),
            pltpu.VMEM_SHARED((N_PAD, L), jnp.float32),
            pltpu.SemaphoreType.DMA((NBUF,)),
            pltpu.SemaphoreType.DMA((NBUF,)),
        ],
        compiler_params=_SC_PARAMS,
    )
    def edge_kernel(y_hbm, ei3, zeros_hbm, out, src_v, dst_v, rows_v,
                    acc_sh, gsem, ssem):
        cid = lax.axis_index("c")
        sid = lax.axis_index("s")
        wid = sid * NC + cid
        pltpu.sync_copy(zeros_hbm, acc_sh.at[pl.ds(sid * RPT, RPT)])
        pltpu.sync_copy(ei3.at[0, pl.ds(wid * k_per_tile, k_per_tile)], src_v)
        pltpu.sync_copy(ei3.at[1, pl.ds(wid * k_per_tile, k_per_tile)], dst_v)
        plsc.subcore_barrier()
        gd = {}
        sd = {}
        for j in range(min(NBUF, k_per_tile)):
            gd[j] = pltpu.async_copy(
                y_hbm.at[src_v.at[j]], rows_v.at[j % NBUF], gsem.at[j % NBUF]
            )
        lag = NBUF // 2
        for j in range(k_per_tile):
            b = j % NBUF
            gd[j].wait()
            sd[j] = pltpu.async_copy(
                rows_v.at[b], acc_sh.at[dst_v.at[j]], ssem.at[b], add=True
            )
            jj = j - lag
            if jj >= 0 and jj + NBUF < k_per_tile:
                nj = jj + NBUF
                sd[jj].wait()
                gd[nj] = pltpu.async_copy(
                    y_hbm.at[src_v.at[nj]], rows_v.at[nj % NBUF],
                    gsem.at[nj % NBUF]
                )
        for j in range(max(k_per_tile - NBUF, 0), k_per_tile):
            sd[j].wait()
        plsc.subcore_barrier()
        pltpu.sync_copy(
            acc_sh.at[pl.ds(sid * RPT, RPT)],
            out.at[cid, pl.ds(sid * RPT, RPT)],
        )

    return edge_kernel


def _block_diag(w, reps):
    r, c = w.shape
    tiled = jnp.tile(w, (reps, reps))
    i0 = lax.broadcasted_iota(jnp.int32, (r * reps, c * reps), 0) // r
    i1 = lax.broadcasted_iota(jnp.int32, (r * reps, c * reps), 1) // c
    return jnp.where(i0 == i1, tiled, 0.0)


def _tca_body(xg_ref, w1_ref, xw_ref):
    bd = _block_diag(w1_ref[...], G)
    xw = jnp.dot(xg_ref[...], bd, preferred_element_type=jnp.float32)
    xw_ref[...] = jnp.concatenate(
        [xw, jnp.zeros((NG - NG_REAL, G * HIDDEN), jnp.float32)], axis=0
    )


def _tcb_body(degp_ref, xw_ref, dis_ref, y1_ref):
    dis = lax.rsqrt(degp_ref[0] + degp_ref[1] + 1.0)
    dis_ref[...] = dis
    y1_ref[...] = dis * xw_ref[...]


def _tc2_body(sp_ref, y1_ref, dis_ref, w2_ref, b1_ref, y2_ref):
    dis = dis_ref[...]
    h1 = jnp.maximum(dis * (sp_ref[0] + sp_ref[1] + y1_ref[...]) + b1_ref[...],
                     0.0)
    bd = _block_diag(w2_ref[...], G)
    y2_ref[...] = dis * jnp.dot(h1, bd, preferred_element_type=jnp.float32)


def _tc3_body(sp_ref, y2_ref, dis_ref, b2_ref, pred_ref, out_ref):
    h2_g = dis_ref[...] * (sp_ref[0] + sp_ref[1] + y2_ref[...]) + b2_ref[...]
    rows = lax.broadcasted_iota(jnp.int32, (N_DRUGS, N_DRUGS // G), 0)
    cols = lax.broadcasted_iota(jnp.int32, (N_DRUGS, N_DRUGS // G), 1)
    h2 = jnp.zeros((N_DRUGS, HIDDEN), jnp.float32)
    for i in range(G):
        e_i = jnp.where((rows % G == i) & (rows // G == cols), 1.0, 0.0)
        h2 = h2 + jnp.dot(e_i, h2_g[:, i * HIDDEN:(i + 1) * HIDDEN],
                          preferred_element_type=jnp.float32)
    t = jnp.dot(h2, pred_ref[...], preferred_element_type=jnp.float32)
    out_ref[...] = lax.dot_general(
        t, h2, (((1,), (1,)), ((), ())), preferred_element_type=jnp.float32
    )


def kernel(x, edge_index, number_of_drugs, W1, b1, W2, b2, predictor):
    del number_of_drugs
    e = edge_index.shape[1]
    k_per_tile = e // (NW * CHUNK)
    ei3 = edge_index.astype(jnp.int32).reshape(2, NW * k_per_tile, CHUNK)

    zeros = jnp.zeros((RPT, L), jnp.float32)
    ones = jnp.ones((CHUNK, L), jnp.float32)
    b1t = jnp.tile(b1, G).reshape(1, G * HIDDEN)
    b2t = jnp.tile(b2, G).reshape(1, G * HIDDEN)
    nd_g = N_DRUGS // G

    deg_kernel = _make_deg_kernel(k_per_tile)
    edge_kernel = _make_edge_kernel(k_per_tile)

    xw_g = pl.pallas_call(
        _tca_body,
        out_shape=jax.ShapeDtypeStruct((NG, G * HIDDEN), jnp.float32),
    )(x.reshape(NG_REAL, G * D_FEAT), W1)

    degp = deg_kernel(ei3, ones, zeros).reshape(NC, NG, G * HIDDEN)

    dis_g, y1_g = pl.pallas_call(
        _tcb_body,
        out_shape=[
            jax.ShapeDtypeStruct((NG, G * HIDDEN), jnp.float32),
            jax.ShapeDtypeStruct((NG, G * HIDDEN), jnp.float32),
        ],
    )(degp, xw_g)

    sp1 = edge_kernel(y1_g.reshape(N_PAD, L), ei3, zeros).reshape(
        NC, NG, G * HIDDEN)

    y2_g = pl.pallas_call(
        _tc2_body,
        out_shape=jax.ShapeDtypeStruct((NG, G * HIDDEN), jnp.float32),
    )(sp1, y1_g, dis_g, W2, b1t)

    sp2 = edge_kernel(y2_g.reshape(N_PAD, L), ei3, zeros)

    sp2_g = sp2.reshape(NC, NG, G * HIDDEN)
    scores = pl.pallas_call(
        _tc3_body,
        out_shape=jax.ShapeDtypeStruct((N_DRUGS, N_DRUGS), jnp.float32),
    )(sp2_g[:, :nd_g], y2_g[:nd_g], dis_g[:nd_g], b2t, predictor)

    return scores
